# Initial kernel scaffold; baseline (speedup 1.0000x reference)
import jax
import jax.numpy as jnp
from jax.experimental import pallas as pl


def kernel(x, edge_index, edge_attr, u, W1, b1, W2, b2, Wd1, bd1, Wd2, bd2, Wd3, bd3):
    raise NotImplementedError("write your pallas kernel here")


# speedup vs baseline: 9.7042x; 9.7042x over previous
import functools

import jax
import jax.numpy as jnp
from jax import lax
from jax.experimental import pallas as pl
from jax.experimental.pallas import tpu as pltpu
from jax.experimental.pallas import tpu_sc as plsc

NC = 2
NS = 16

_MESH = plsc.VectorSubcoreMesh(
    core_axis_name="c", subcore_axis_name="s", num_cores=NC, num_subcores=NS)


def _fill_f32(ref, n, value):
    vec = jnp.full((16,), value, jnp.float32)

    def body(j, carry):
        ref[pl.ds(j * 16, 16)] = vec
        return carry

    lax.fori_loop(0, n // 16, body, 0)
    if n % 16:
        ref[pl.ds(n - 16, 16)] = vec


def _degree(dst, n, e):
    ew = e // (NC * NS)
    c = 5000
    zc = 1000
    assert ew % c == 0 and n % zc == 0

    @functools.partial(
        pl.kernel,
        out_type=jax.ShapeDtypeStruct((NC, n), jnp.float32),
        mesh=_MESH,
        compiler_params=pltpu.CompilerParams(use_tc_tiling_on_sc=False),
        scratch_types=[
            pltpu.VMEM_SHARED((n,), jnp.float32),
            pltpu.VMEM((c,), jnp.int32),
            pltpu.VMEM((c,), jnp.float32),
            pltpu.VMEM((zc,), jnp.float32),
            pltpu.SemaphoreType.DMA,
        ],
    )
    def k(dst_hbm, out_hbm, deg_sh, idx_v, ones_v, zeros_v, sem):
        core = lax.axis_index("c")
        sid = lax.axis_index("s")
        wid = core * NS + sid
        _fill_f32(ones_v, c, 1.0)
        _fill_f32(zeros_v, zc, 0.0)

        @pl.when(sid == 0)
        def _():
            def zbody(i, carry):
                pltpu.sync_copy(zeros_v, deg_sh.at[pl.ds(i * zc, zc)])
                return carry
            lax.fori_loop(0, n // zc, zbody, 0)

        plsc.subcore_barrier()

        def ebody(i, carry):
            base = wid * ew + i * c
            pltpu.sync_copy(dst_hbm.at[pl.ds(base, c)], idx_v)
            pltpu.sync_copy(ones_v, deg_sh.at[idx_v], add=True)
            return carry

        lax.fori_loop(0, ew // c, ebody, 0)
        plsc.subcore_barrier()

        @pl.when((sid == 0) & (core == 0))
        def _():
            pltpu.sync_copy(deg_sh, out_hbm.at[0])

        @pl.when((sid == 0) & (core == 1))
        def _():
            pltpu.sync_copy(deg_sh, out_hbm.at[1])

    return k(dst)


def _gcn_scatter(src, dst, ga, gb, n, e):
    h = ga.shape[1]
    et = e // NS
    c = 400
    assert et % c == 0

    @functools.partial(
        pl.kernel,
        out_type=jax.ShapeDtypeStruct((NC, n, h), jnp.float32),
        mesh=_MESH,
        compiler_params=pltpu.CompilerParams(use_tc_tiling_on_sc=False),
        scratch_types=[
            pltpu.VMEM_SHARED((n, h), jnp.float32),
            pltpu.VMEM((c,), jnp.int32),
            pltpu.VMEM((c,), jnp.int32),
            pltpu.VMEM((c, h), jnp.float32),
            pltpu.SemaphoreType.DMA,
        ],
    )
    def k(src_hbm, dst_hbm, ga_hbm, gb_hbm, out_hbm, s_sh, idxs_v, idxd_v,
          rows_v, sem):
        core = lax.axis_index("c")
        sid = lax.axis_index("s")

        @pl.when((sid == 0) & (core == 0))
        def _():
            pltpu.sync_copy(ga_hbm, s_sh)

        @pl.when((sid == 0) & (core == 1))
        def _():
            pltpu.sync_copy(gb_hbm, s_sh)

        plsc.subcore_barrier()

        def ebody(i, carry):
            base = sid * et + i * c
            pltpu.sync_copy(src_hbm.at[pl.ds(base, c)], idxs_v)
            pltpu.sync_copy(dst_hbm.at[pl.ds(base, c)], idxd_v)

            @pl.when(core == 0)
            def _():
                pltpu.async_copy(ga_hbm.at[idxs_v], rows_v, sem).wait()

            @pl.when(core == 1)
            def _():
                pltpu.async_copy(gb_hbm.at[idxs_v], rows_v, sem).wait()

            pltpu.sync_copy(rows_v, s_sh.at[idxd_v], add=True)
            return carry

        lax.fori_loop(0, et // c, ebody, 0)
        plsc.subcore_barrier()

        @pl.when((sid == 0) & (core == 0))
        def _():
            pltpu.sync_copy(s_sh, out_hbm.at[0])

        @pl.when((sid == 0) & (core == 1))
        def _():
            pltpu.sync_copy(s_sh, out_hbm.at[1])

    return k(src, dst, ga, gb)


def _edge_gather(src, dst, a, b, e):
    h = a.shape[1]
    ew = e // (NC * NS)
    c = 1000
    assert ew % c == 0

    @functools.partial(
        pl.kernel,
        out_type=(jax.ShapeDtypeStruct((e, h), jnp.float32),
                  jax.ShapeDtypeStruct((e, h), jnp.float32)),
        mesh=_MESH,
        compiler_params=pltpu.CompilerParams(use_tc_tiling_on_sc=False),
        scratch_types=[
            pltpu.VMEM((c,), jnp.int32),
            pltpu.VMEM((c, h), jnp.float32),
            pltpu.SemaphoreType.DMA,
        ],
    )
    def k(src_hbm, dst_hbm, a_hbm, b_hbm, asrc_hbm, bdst_hbm, idx_v, rows_v,
          sem):
        core = lax.axis_index("c")
        sid = lax.axis_index("s")
        wid = core * NS + sid

        def ebody(i, carry):
            base = wid * ew + i * c
            pltpu.sync_copy(src_hbm.at[pl.ds(base, c)], idx_v)
            pltpu.async_copy(a_hbm.at[idx_v], rows_v, sem).wait()
            pltpu.sync_copy(rows_v, asrc_hbm.at[pl.ds(base, c)])
            pltpu.sync_copy(dst_hbm.at[pl.ds(base, c)], idx_v)
            pltpu.async_copy(b_hbm.at[idx_v], rows_v, sem).wait()
            pltpu.sync_copy(rows_v, bdst_hbm.at[pl.ds(base, c)])
            return carry

        lax.fori_loop(0, ew // c, ebody, 0)

    return k(src, dst, a, b)


_BN = 1024


def _tc_layer1(x, degt, u, w1a, w1b):
    n = x.shape[0]

    def body(x_ref, deg_ref, u_ref, wa_ref, wb_ref, ga_ref, gb_ref):
        deg = deg_ref[:, 0:1] + deg_ref[:, 1:2] + 1.0
        dinv = lax.rsqrt(deg)
        hw = jnp.dot(x_ref[...], wa_ref[...], preferred_element_type=jnp.float32)
        hw = hw + jnp.dot(u_ref[...], wb_ref[...],
                          preferred_element_type=jnp.float32)
        g = hw * dinv
        ga_ref[...] = g[:, :32]
        gb_ref[...] = g[:, 32:]

    return pl.pallas_call(
        body,
        grid=(pl.cdiv(n, _BN),),
        in_specs=[
            pl.BlockSpec((_BN, x.shape[1]), lambda i: (i, 0)),
            pl.BlockSpec((_BN, 2), lambda i: (i, 0)),
            pl.BlockSpec(u.shape, lambda i: (0, 0)),
            pl.BlockSpec(w1a.shape, lambda i: (0, 0)),
            pl.BlockSpec(w1b.shape, lambda i: (0, 0)),
        ],
        out_specs=[pl.BlockSpec((_BN, 32), lambda i: (i, 0))] * 2,
        out_shape=[jax.ShapeDtypeStruct((n, 32), jnp.float32)] * 2,
    )(x, degt, u, w1a, w1b)


def _tc_layer2(s1, degt, b1, w2):
    n = degt.shape[0]

    def body(s_ref, deg_ref, b_ref, w_ref, ga_ref, gb_ref):
        deg = deg_ref[:, 0:1] + deg_ref[:, 1:2] + 1.0
        dinv = lax.rsqrt(deg)
        s = jnp.concatenate([s_ref[0], s_ref[1]], axis=1)
        h1 = jnp.maximum(s * dinv + b_ref[...], 0.0)
        g = jnp.dot(h1, w_ref[...], preferred_element_type=jnp.float32) * dinv
        ga_ref[...] = g[:, :32]
        gb_ref[...] = g[:, 32:]

    return pl.pallas_call(
        body,
        grid=(pl.cdiv(n, _BN),),
        in_specs=[
            pl.BlockSpec((2, _BN, 32), lambda i: (0, i, 0)),
            pl.BlockSpec((_BN, 2), lambda i: (i, 0)),
            pl.BlockSpec(b1.shape, lambda i: (0, 0)),
            pl.BlockSpec(w2.shape, lambda i: (0, 0)),
        ],
        out_specs=[pl.BlockSpec((_BN, 32), lambda i: (i, 0))] * 2,
        out_shape=[jax.ShapeDtypeStruct((n, 32), jnp.float32)] * 2,
    )(s1, degt, b1, w2)


def _tc_layer3(s2, degt, b2, wd1s, wd1d):
    n = degt.shape[0]

    def body(s_ref, deg_ref, b_ref, ws_ref, wd_ref, a_ref, bb_ref):
        deg = deg_ref[:, 0:1] + deg_ref[:, 1:2] + 1.0
        dinv = lax.rsqrt(deg)
        s = jnp.concatenate([s_ref[0], s_ref[1]], axis=1)
        h2 = jnp.maximum(s * dinv + b_ref[...], 0.0)
        a_ref[...] = jnp.dot(h2, ws_ref[...],
                             preferred_element_type=jnp.float32)
        bb_ref[...] = jnp.dot(h2, wd_ref[...],
                              preferred_element_type=jnp.float32)

    return pl.pallas_call(
        body,
        grid=(pl.cdiv(n, _BN),),
        in_specs=[
            pl.BlockSpec((2, _BN, 32), lambda i: (0, i, 0)),
            pl.BlockSpec((_BN, 2), lambda i: (i, 0)),
            pl.BlockSpec(b2.shape, lambda i: (0, 0)),
            pl.BlockSpec(wd1s.shape, lambda i: (0, 0)),
            pl.BlockSpec(wd1d.shape, lambda i: (0, 0)),
        ],
        out_specs=[pl.BlockSpec((_BN, 64), lambda i: (i, 0))] * 2,
        out_shape=[jax.ShapeDtypeStruct((n, 64), jnp.float32)] * 2,
    )(s2, degt, b2, wd1s, wd1d)


def _tc_decoder(asrc, bdst, ea, wd1e, bd1, wd2, bd2, wd3, bd3):
    e = asrc.shape[0]
    be = 4000

    def body(a_ref, b_ref, e_ref, w1_ref, b1_ref, w2_ref, b2_ref, w3_ref,
             b3_ref, o_ref):
        z = a_ref[...] + b_ref[...] + b1_ref[...]
        z = z + jnp.dot(e_ref[...], w1_ref[...],
                        preferred_element_type=jnp.float32)
        z = jnp.maximum(z, 0.0)
        z = jnp.maximum(
            jnp.dot(z, w2_ref[...], preferred_element_type=jnp.float32)
            + b2_ref[...], 0.0)
        o_ref[...] = jnp.dot(z, w3_ref[...],
                             preferred_element_type=jnp.float32) + b3_ref[...]

    return pl.pallas_call(
        body,
        grid=(pl.cdiv(e, be),),
        in_specs=[
            pl.BlockSpec((be, 64), lambda i: (i, 0)),
            pl.BlockSpec((be, 64), lambda i: (i, 0)),
            pl.BlockSpec((be, ea.shape[1]), lambda i: (i, 0)),
            pl.BlockSpec(wd1e.shape, lambda i: (0, 0)),
            pl.BlockSpec(bd1.shape, lambda i: (0, 0)),
            pl.BlockSpec(wd2.shape, lambda i: (0, 0)),
            pl.BlockSpec(bd2.shape, lambda i: (0, 0)),
            pl.BlockSpec(wd3.shape, lambda i: (0, 0)),
            pl.BlockSpec(bd3.shape, lambda i: (0, 0)),
        ],
        out_specs=pl.BlockSpec((be, wd3.shape[1]), lambda i: (i, 0)),
        out_shape=jax.ShapeDtypeStruct((e, wd3.shape[1]), jnp.float32),
    )(asrc, bdst, ea, wd1e, bd1, wd2, bd2, wd3, bd3)


def kernel(x, edge_index, edge_attr, u, W1, b1, W2, b2, Wd1, bd1, Wd2, bd2,
           Wd3, bd3):
    n = x.shape[0]
    e = edge_index.shape[1]
    nd = x.shape[1]
    hid = W2.shape[0]
    src = edge_index[0].astype(jnp.int32)
    dst = edge_index[1].astype(jnp.int32)

    deg2 = _degree(dst, n, e)
    degt = deg2.T

    ga1, gb1 = _tc_layer1(x, degt, u, W1[:nd], W1[nd:])
    s1 = _gcn_scatter(src, dst, ga1, gb1, n, e)
    ga2, gb2 = _tc_layer2(s1, degt, b1.reshape(1, -1), W2)
    s2 = _gcn_scatter(src, dst, ga2, gb2, n, e)
    a, b = _tc_layer3(s2, degt, b2.reshape(1, -1), Wd1[:hid], Wd1[hid:2 * hid])
    asrc, bdst = _edge_gather(src, dst, a, b, e)
    return _tc_decoder(asrc, bdst, edge_attr, Wd1[2 * hid:],
                       bd1.reshape(1, -1), Wd2, bd2.reshape(1, -1), Wd3,
                       bd3.reshape(1, -1))
